# initial kernel scaffold (unmeasured)
import jax
import jax.numpy as jnp
from jax import lax
from jax.experimental import pallas as pl
from jax.experimental.pallas import tpu as pltpu

B, S, D = 4, 256, 4096
H, Dh, Dr = 32, 128, 64
DC_HALF = 128
SCALE = (Dh + Dr) ** -0.5


def _mm_body(x_ref, w_ref, o_ref, acc_ref):
    @pl.when(pl.program_id(2) == 0)
    def _():
        acc_ref[...] = jnp.zeros_like(acc_ref)

    acc_ref[...] += jnp.dot(
        x_ref[...], w_ref[...], preferred_element_type=jnp.float32
    )

    @pl.when(pl.program_id(2) == pl.num_programs(2) - 1)
    def _():
        o_ref[...] = acc_ref[...]


def _matmul(x, w, bm=1024, bn=1024, bk=1024):
    m, k = x.shape
    _, n = w.shape
    bm, bn, bk = min(bm, m), min(bn, n), min(bk, k)
    return pl.pallas_call(
        _mm_body,
        grid=(m // bm, n // bn, k // bk),
        in_specs=[
            pl.BlockSpec((bm, bk), lambda i, j, kk: (i, kk)),
            pl.BlockSpec((bk, bn), lambda i, j, kk: (kk, j)),
        ],
        out_specs=pl.BlockSpec((bm, bn), lambda i, j, kk: (i, j)),
        out_shape=jax.ShapeDtypeStruct((m, n), jnp.float32),
        scratch_shapes=[pltpu.VMEM((bm, bn), jnp.float32)],
    )(x, w)


def _gather_body(c_ref, wuk_ref, wuv_ref,
                 co_ref, wuko_ref, wuvo_ref,
                 send_sems, recv_sems):
    my_x = lax.axis_index("x")
    my_y = lax.axis_index("y")
    nbr = (my_x, 1 - my_y)

    barrier = pltpu.get_barrier_semaphore()
    pl.semaphore_signal(
        barrier, inc=1, device_id=nbr, device_id_type=pl.DeviceIdType.MESH
    )
    pl.semaphore_wait(barrier, 1)

    off = my_y * DC_HALF
    copies = [
        pltpu.make_async_remote_copy(
            src_ref=c_ref,
            dst_ref=co_ref.at[:, pl.ds(off, DC_HALF)],
            send_sem=send_sems.at[0],
            recv_sem=recv_sems.at[0],
            device_id=nbr,
            device_id_type=pl.DeviceIdType.MESH,
        ),
        pltpu.make_async_remote_copy(
            src_ref=wuk_ref,
            dst_ref=wuko_ref.at[pl.ds(off, DC_HALF), :],
            send_sem=send_sems.at[1],
            recv_sem=recv_sems.at[1],
            device_id=nbr,
            device_id_type=pl.DeviceIdType.MESH,
        ),
        pltpu.make_async_remote_copy(
            src_ref=wuv_ref,
            dst_ref=wuvo_ref.at[pl.ds(off, DC_HALF), :],
            send_sem=send_sems.at[2],
            recv_sem=recv_sems.at[2],
            device_id=nbr,
            device_id_type=pl.DeviceIdType.MESH,
        ),
    ]
    for cp in copies:
        cp.start()

    co_ref[:, pl.ds(off, DC_HALF)] = c_ref[...]
    wuko_ref[pl.ds(off, DC_HALF), :] = wuk_ref[...]
    wuvo_ref[pl.ds(off, DC_HALF), :] = wuv_ref[...]

    for cp in copies:
        cp.wait()


def _y_allgather(c_loc, Wuk, Wuv):
    return pl.pallas_call(
        _gather_body,
        out_shape=(
            jax.ShapeDtypeStruct((B * S, 2 * DC_HALF), jnp.float32),
            jax.ShapeDtypeStruct((2 * DC_HALF, H * Dh), jnp.float32),
            jax.ShapeDtypeStruct((2 * DC_HALF, H * Dh), jnp.float32),
        ),
        in_specs=[pl.BlockSpec(memory_space=pltpu.VMEM)] * 3,
        out_specs=[pl.BlockSpec(memory_space=pltpu.VMEM)] * 3,
        scratch_shapes=[
            pltpu.SemaphoreType.DMA((3,)),
            pltpu.SemaphoreType.DMA((3,)),
        ],
        compiler_params=pltpu.CompilerParams(collective_id=0),
    )(c_loc, Wuk, Wuv)


def _attn_body(q_ref, k_ref, v_ref, qr_ref, kr_ref, o_ref):
    q = q_ref[0, :, 0, :]
    k = k_ref[0, :, 0, :]
    v = v_ref[0, :, 0, :]
    qr = qr_ref[0, :, 0, :]
    kr = kr_ref[0, :, :]
    s = (
        lax.dot_general(q, k, (((1,), (1,)), ((), ())),
                        preferred_element_type=jnp.float32)
        + lax.dot_general(qr, kr, (((1,), (1,)), ((), ())),
                          preferred_element_type=jnp.float32)
    ) * SCALE
    m = jnp.max(s, axis=-1, keepdims=True)
    p = jnp.exp(s - m)
    p = p / jnp.sum(p, axis=-1, keepdims=True)
    o_ref[0, :, 0, :] = jnp.dot(p, v, preferred_element_type=jnp.float32)


def _attention(Q4, K4, V4, Qr4, Kr3):
    b = Q4.shape[0]
    return pl.pallas_call(
        _attn_body,
        grid=(b, H),
        in_specs=[
            pl.BlockSpec((1, S, 1, Dh), lambda i, h: (i, 0, h, 0)),
            pl.BlockSpec((1, S, 1, Dh), lambda i, h: (i, 0, h, 0)),
            pl.BlockSpec((1, S, 1, Dh), lambda i, h: (i, 0, h, 0)),
            pl.BlockSpec((1, S, 1, Dh), lambda i, h: (i, 0, h, 0)),
            pl.BlockSpec((1, S, Dh), lambda i, h: (i, 0, 0)),
        ],
        out_specs=pl.BlockSpec((1, S, 1, Dh), lambda i, h: (i, 0, h, 0)),
        out_shape=jax.ShapeDtypeStruct((b, S, H, Dh), jnp.float32),
    )(Q4, K4, V4, Qr4, Kr3)


def kernel(x, Wdkv, Wuk, Wuv, Wq, Wqr, Wkr, Wo):
    x2 = x.reshape(B * S, D)

    c_loc = _matmul(x2, Wdkv)
    c_full, Wuk_f, Wuv_f = _y_allgather(c_loc, Wuk, Wuv)

    K = _matmul(c_full, Wuk_f)
    V = _matmul(c_full, Wuv_f)
    Q = _matmul(x2, Wq)
    Qr = _matmul(x2, Wqr)
    Wkr_p = jnp.pad(Wkr, ((0, 0), (0, Dh - Dr)))
    Kr = _matmul(x2, Wkr_p)

    Q4 = Q.reshape(B, S, H, Dh)
    K4 = K.reshape(B, S, H, Dh)
    V4 = V.reshape(B, S, H, Dh)
    Qr4 = jnp.pad(
        Qr.reshape(B, S, H, Dr), ((0, 0), (0, 0), (0, 0), (0, Dh - Dr))
    )
    Kr3 = Kr.reshape(B, S, Dh)

    O = _attention(Q4, K4, V4, Qr4, Kr3)
    out = _matmul(O.reshape(B * S, H * Dh), Wo)
    return out.reshape(B, S, D)


# baseline (device time: 457748 ns/iter reference)
import jax
import jax.numpy as jnp
from jax import lax
from jax.experimental import pallas as pl
from jax.experimental.pallas import tpu as pltpu

B, S, D = 4, 256, 4096
H, Dh, Dr = 32, 128, 64
DC_HALF = 128
SCALE = (Dh + Dr) ** -0.5


def _mm_body(x_ref, w_ref, o_ref, acc_ref):
    @pl.when(pl.program_id(2) == 0)
    def _():
        acc_ref[...] = jnp.zeros_like(acc_ref)

    acc_ref[...] += jnp.dot(
        x_ref[...], w_ref[...], preferred_element_type=jnp.float32
    )

    @pl.when(pl.program_id(2) == pl.num_programs(2) - 1)
    def _():
        o_ref[...] = acc_ref[...]


def _matmul(x, w, bm=1024, bn=1024, bk=1024):
    m, k = x.shape
    _, n = w.shape
    bm, bn, bk = min(bm, m), min(bn, n), min(bk, k)
    return pl.pallas_call(
        _mm_body,
        grid=(m // bm, n // bn, k // bk),
        in_specs=[
            pl.BlockSpec((bm, bk), lambda i, j, kk: (i, kk)),
            pl.BlockSpec((bk, bn), lambda i, j, kk: (kk, j)),
        ],
        out_specs=pl.BlockSpec((bm, bn), lambda i, j, kk: (i, j)),
        out_shape=jax.ShapeDtypeStruct((m, n), jnp.float32),
        scratch_shapes=[pltpu.VMEM((bm, bn), jnp.float32)],
    )(x, w)


def _gather_body(c_ref, wuk_ref, wuv_ref,
                 co_ref, wuko_ref, wuvo_ref,
                 send_sems, recv_sems):
    my_x = lax.axis_index("x")
    my_y = lax.axis_index("y")
    nbr = (my_x, 1 - my_y)

    barrier = pltpu.get_barrier_semaphore()
    pl.semaphore_signal(
        barrier, inc=1, device_id=nbr, device_id_type=pl.DeviceIdType.MESH
    )
    pl.semaphore_wait(barrier, 1)

    off = my_y * DC_HALF
    copies = [
        pltpu.make_async_remote_copy(
            src_ref=c_ref,
            dst_ref=co_ref.at[:, pl.ds(off, DC_HALF)],
            send_sem=send_sems.at[0],
            recv_sem=recv_sems.at[0],
            device_id=nbr,
            device_id_type=pl.DeviceIdType.MESH,
        ),
        pltpu.make_async_remote_copy(
            src_ref=wuk_ref,
            dst_ref=wuko_ref.at[pl.ds(off, DC_HALF), :],
            send_sem=send_sems.at[1],
            recv_sem=recv_sems.at[1],
            device_id=nbr,
            device_id_type=pl.DeviceIdType.MESH,
        ),
        pltpu.make_async_remote_copy(
            src_ref=wuv_ref,
            dst_ref=wuvo_ref.at[pl.ds(off, DC_HALF), :],
            send_sem=send_sems.at[2],
            recv_sem=recv_sems.at[2],
            device_id=nbr,
            device_id_type=pl.DeviceIdType.MESH,
        ),
    ]
    for cp in copies:
        cp.start()

    co_ref[:, pl.ds(off, DC_HALF)] = c_ref[...]
    wuko_ref[pl.ds(off, DC_HALF), :] = wuk_ref[...]
    wuvo_ref[pl.ds(off, DC_HALF), :] = wuv_ref[...]

    for cp in copies:
        cp.wait()


def _y_allgather(c_loc, Wuk, Wuv):
    return pl.pallas_call(
        _gather_body,
        out_shape=(
            jax.ShapeDtypeStruct((B * S, 2 * DC_HALF), jnp.float32),
            jax.ShapeDtypeStruct((2 * DC_HALF, H * Dh), jnp.float32),
            jax.ShapeDtypeStruct((2 * DC_HALF, H * Dh), jnp.float32),
        ),
        in_specs=[pl.BlockSpec(memory_space=pltpu.VMEM)] * 3,
        out_specs=[pl.BlockSpec(memory_space=pltpu.VMEM)] * 3,
        scratch_shapes=[
            pltpu.SemaphoreType.DMA((3,)),
            pltpu.SemaphoreType.DMA((3,)),
        ],
        compiler_params=pltpu.CompilerParams(collective_id=0),
    )(c_loc, Wuk, Wuv)


def _attn_body(q_ref, k_ref, v_ref, qr_ref, kr_ref, o_ref):
    q = q_ref[0, 0, :, :]
    k = k_ref[0, 0, :, :]
    v = v_ref[0, 0, :, :]
    qr = qr_ref[0, 0, :, :]
    kr = kr_ref[0, :, :]
    s = (
        lax.dot_general(q, k, (((1,), (1,)), ((), ())),
                        preferred_element_type=jnp.float32)
        + lax.dot_general(qr, kr, (((1,), (1,)), ((), ())),
                          preferred_element_type=jnp.float32)
    ) * SCALE
    m = jnp.max(s, axis=-1, keepdims=True)
    p = jnp.exp(s - m)
    p = p / jnp.sum(p, axis=-1, keepdims=True)
    o_ref[0, 0, :, :] = jnp.dot(p, v, preferred_element_type=jnp.float32)


def _attention(Q4, K4, V4, Qr4, Kr3):
    b = Q4.shape[0]
    hs = pl.BlockSpec((1, 1, S, Dh), lambda i, h: (i, h, 0, 0))
    return pl.pallas_call(
        _attn_body,
        grid=(b, H),
        in_specs=[hs, hs, hs, hs,
                  pl.BlockSpec((1, S, Dh), lambda i, h: (i, 0, 0))],
        out_specs=pl.BlockSpec((1, 1, S, Dh), lambda i, h: (i, h, 0, 0)),
        out_shape=jax.ShapeDtypeStruct((b, H, S, Dh), jnp.float32),
    )(Q4, K4, V4, Qr4, Kr3)


def kernel(x, Wdkv, Wuk, Wuv, Wq, Wqr, Wkr, Wo):
    x2 = x.reshape(B * S, D)

    c_loc = _matmul(x2, Wdkv)
    c_full, Wuk_f, Wuv_f = _y_allgather(c_loc, Wuk, Wuv)

    K = _matmul(c_full, Wuk_f)
    V = _matmul(c_full, Wuv_f)
    Q = _matmul(x2, Wq)
    Qr = _matmul(x2, Wqr)
    Wkr_p = jnp.pad(Wkr, ((0, 0), (0, Dh - Dr)))
    Kr = _matmul(x2, Wkr_p)

    Q4 = Q.reshape(B, S, H, Dh).transpose(0, 2, 1, 3)
    K4 = K.reshape(B, S, H, Dh).transpose(0, 2, 1, 3)
    V4 = V.reshape(B, S, H, Dh).transpose(0, 2, 1, 3)
    Qr4 = jnp.pad(
        Qr.reshape(B, S, H, Dr), ((0, 0), (0, 0), (0, 0), (0, Dh - Dr))
    ).transpose(0, 2, 1, 3)
    Kr3 = Kr.reshape(B, S, Dh)

    O = _attention(Q4, K4, V4, Qr4, Kr3)
    O2 = O.transpose(0, 2, 1, 3).reshape(B * S, H * Dh)
    out = _matmul(O2, Wo)
    return out.reshape(B, S, D)


# device time: 260631 ns/iter; 1.7563x vs baseline; 1.7563x over previous
import jax
import jax.numpy as jnp
from jax import lax
from jax.experimental import pallas as pl
from jax.experimental.pallas import tpu as pltpu

B, S, D = 4, 256, 4096
H, Dh, Dr = 32, 128, 64
DC_HALF = 128
SCALE = (Dh + Dr) ** -0.5
_MESH = pl.DeviceIdType.MESH


def _mm_body(x_ref, w_ref, o_ref, acc_ref):
    @pl.when(pl.program_id(2) == 0)
    def _():
        acc_ref[...] = jnp.zeros_like(acc_ref)

    acc_ref[...] += jnp.dot(
        x_ref[...], w_ref[...], preferred_element_type=jnp.float32
    )

    @pl.when(pl.program_id(2) == pl.num_programs(2) - 1)
    def _():
        o_ref[...] = acc_ref[...]


def _matmul(x, w, bm=1024, bn=1024, bk=1024):
    m, k = x.shape
    _, n = w.shape
    bm, bn, bk = min(bm, m), min(bn, n), min(bk, k)
    return pl.pallas_call(
        _mm_body,
        grid=(m // bm, n // bn, k // bk),
        in_specs=[
            pl.BlockSpec((bm, bk), lambda i, j, kk: (i, kk)),
            pl.BlockSpec((bk, bn), lambda i, j, kk: (kk, j)),
        ],
        out_specs=pl.BlockSpec((bm, bn), lambda i, j, kk: (i, j)),
        out_shape=jax.ShapeDtypeStruct((m, n), jnp.float32),
        scratch_shapes=[pltpu.VMEM((bm, bn), jnp.float32)],
    )(x, w)


def _gather_body(cp_ref, wuk_ref, wuv_ref,
                 co_ref, wuko_ref, wuvo_ref,
                 send_sems, recv_sems):
    my_x = lax.axis_index("x")
    my_y = lax.axis_index("y")
    nbr = (my_x, 1 - my_y)
    q_me = jnp.where(my_x == 0, my_y, 1 - my_y)
    q_nbr = 1 - q_me

    barrier = pltpu.get_barrier_semaphore()
    pl.semaphore_signal(barrier, inc=1, device_id=nbr, device_id_type=_MESH)
    pl.semaphore_wait(barrier, 1)

    off = my_y * DC_HALF
    copies = [
        pltpu.make_async_remote_copy(
            src_ref=cp_ref.at[q_nbr],
            dst_ref=co_ref.at[:, pl.ds(off, DC_HALF)],
            send_sem=send_sems.at[0], recv_sem=recv_sems.at[0],
            device_id=nbr, device_id_type=_MESH,
        ),
        pltpu.make_async_remote_copy(
            src_ref=wuk_ref,
            dst_ref=wuko_ref.at[pl.ds(off, DC_HALF), :],
            send_sem=send_sems.at[1], recv_sem=recv_sems.at[1],
            device_id=nbr, device_id_type=_MESH,
        ),
        pltpu.make_async_remote_copy(
            src_ref=wuv_ref,
            dst_ref=wuvo_ref.at[pl.ds(off, DC_HALF), :],
            send_sem=send_sems.at[2], recv_sem=recv_sems.at[2],
            device_id=nbr, device_id_type=_MESH,
        ),
    ]
    for cp in copies:
        cp.start()

    co_ref[:, pl.ds(off, DC_HALF)] = cp_ref[q_me]
    wuko_ref[pl.ds(off, DC_HALF), :] = wuk_ref[...]
    wuvo_ref[pl.ds(off, DC_HALF), :] = wuv_ref[...]

    for cp in copies:
        cp.wait()


def _y_exchange(c_pair, Wuk, Wuv):
    return pl.pallas_call(
        _gather_body,
        out_shape=(
            jax.ShapeDtypeStruct((S, 2 * DC_HALF), jnp.float32),
            jax.ShapeDtypeStruct((2 * DC_HALF, H * Dh), jnp.float32),
            jax.ShapeDtypeStruct((2 * DC_HALF, H * Dh), jnp.float32),
        ),
        in_specs=[pl.BlockSpec(memory_space=pltpu.VMEM)] * 3,
        out_specs=[pl.BlockSpec(memory_space=pltpu.VMEM)] * 3,
        scratch_shapes=[
            pltpu.SemaphoreType.DMA((3,)),
            pltpu.SemaphoreType.DMA((3,)),
        ],
        compiler_params=pltpu.CompilerParams(collective_id=0),
    )(c_pair, Wuk, Wuv)


def _ag_body(in_ref, out_ref, send_sems, recv_sems):
    my_x = lax.axis_index("x")
    my_y = lax.axis_index("y")
    p = jnp.where(my_x == 0, my_y, 3 - my_y)
    even = (my_x + my_y) % 2 == 0
    right = (jnp.where(even, my_x, 1 - my_x), jnp.where(even, 1 - my_y, my_y))
    left = (jnp.where(even, 1 - my_x, my_x), jnp.where(even, my_y, 1 - my_y))
    p_left = (p + 3) % 4
    p_right = (p + 1) % 4
    p_diag = (p + 2) % 4
    HALF = S // 2

    barrier = pltpu.get_barrier_semaphore()
    for nb in (left, right):
        pl.semaphore_signal(barrier, inc=1, device_id=nb, device_id_type=_MESH)
    pl.semaphore_wait(barrier, 2)

    out_ref[pl.ds(p * S, S), :] = in_ref[...]

    h1l = pltpu.make_async_remote_copy(
        src_ref=in_ref, dst_ref=out_ref.at[pl.ds(p * S, S), :],
        send_sem=send_sems.at[0], recv_sem=recv_sems.at[0],
        device_id=left, device_id_type=_MESH,
    )
    h1r = pltpu.make_async_remote_copy(
        src_ref=in_ref, dst_ref=out_ref.at[pl.ds(p * S, S), :],
        send_sem=send_sems.at[1], recv_sem=recv_sems.at[1],
        device_id=right, device_id_type=_MESH,
    )
    h1l.start()
    h1r.start()

    h1l.wait_recv()
    h2l = pltpu.make_async_remote_copy(
        src_ref=out_ref.at[pl.ds(p_right * S, HALF), :],
        dst_ref=out_ref.at[pl.ds(p_right * S, HALF), :],
        send_sem=send_sems.at[2], recv_sem=recv_sems.at[2],
        device_id=left, device_id_type=_MESH,
    )
    h2l.start()
    h1r.wait_recv()
    h2r = pltpu.make_async_remote_copy(
        src_ref=out_ref.at[pl.ds(p_left * S + HALF, HALF), :],
        dst_ref=out_ref.at[pl.ds(p_left * S + HALF, HALF), :],
        send_sem=send_sems.at[3], recv_sem=recv_sems.at[3],
        device_id=right, device_id_type=_MESH,
    )
    h2r.start()

    h2l.wait_recv()
    h2r.wait_recv()
    h1l.wait_send()
    h1r.wait_send()
    h2l.wait_send()
    h2r.wait_send()


def _out_allgather(out_me):
    return pl.pallas_call(
        _ag_body,
        out_shape=jax.ShapeDtypeStruct((B * S, D), jnp.float32),
        in_specs=[pl.BlockSpec(memory_space=pltpu.VMEM)],
        out_specs=pl.BlockSpec(memory_space=pltpu.VMEM),
        scratch_shapes=[
            pltpu.SemaphoreType.DMA((4,)),
            pltpu.SemaphoreType.DMA((4,)),
        ],
        compiler_params=pltpu.CompilerParams(collective_id=1),
    )(out_me)


def _attn_body(q_ref, k_ref, v_ref, qr_ref, kr_ref, o_ref):
    q = q_ref[0, 0, :, :]
    k = k_ref[0, 0, :, :]
    v = v_ref[0, 0, :, :]
    qr = qr_ref[0, 0, :, :]
    kr = kr_ref[0, :, :]
    s = (
        lax.dot_general(q, k, (((1,), (1,)), ((), ())),
                        preferred_element_type=jnp.float32)
        + lax.dot_general(qr, kr, (((1,), (1,)), ((), ())),
                          preferred_element_type=jnp.float32)
    ) * SCALE
    m = jnp.max(s, axis=-1, keepdims=True)
    pr = jnp.exp(s - m)
    pr = pr / jnp.sum(pr, axis=-1, keepdims=True)
    o_ref[0, 0, :, :] = jnp.dot(pr, v, preferred_element_type=jnp.float32)


def _attention(Q4, K4, V4, Qr4, Kr3):
    b = Q4.shape[0]
    hs = pl.BlockSpec((1, 1, S, Dh), lambda i, h: (i, h, 0, 0))
    return pl.pallas_call(
        _attn_body,
        grid=(b, H),
        in_specs=[hs, hs, hs, hs,
                  pl.BlockSpec((1, S, Dh), lambda i, h: (i, 0, 0))],
        out_specs=pl.BlockSpec((1, 1, S, Dh), lambda i, h: (i, h, 0, 0)),
        out_shape=jax.ShapeDtypeStruct((b, H, S, Dh), jnp.float32),
    )(Q4, K4, V4, Qr4, Kr3)


def kernel(x, Wdkv, Wuk, Wuv, Wq, Wqr, Wkr, Wo):
    my_x = lax.axis_index("x")
    my_y = lax.axis_index("y")
    p = jnp.where(my_x == 0, my_y, 3 - my_y)

    x_pair = lax.dynamic_slice(x, (2 * my_x, 0, 0), (2, S, D))
    c_pair = _matmul(x_pair.reshape(2 * S, D), Wdkv).reshape(2, S, DC_HALF)
    c_me, Wuk_f, Wuv_f = _y_exchange(c_pair, Wuk, Wuv)

    x_me = lax.dynamic_slice(x, (p, 0, 0), (1, S, D)).reshape(S, D)
    K = _matmul(c_me, Wuk_f)
    V = _matmul(c_me, Wuv_f)
    Q = _matmul(x_me, Wq)
    Qr = _matmul(x_me, Wqr)
    Wkr_p = jnp.pad(Wkr, ((0, 0), (0, Dh - Dr)))
    Kr = _matmul(x_me, Wkr_p)

    Q4 = Q.reshape(1, S, H, Dh).transpose(0, 2, 1, 3)
    K4 = K.reshape(1, S, H, Dh).transpose(0, 2, 1, 3)
    V4 = V.reshape(1, S, H, Dh).transpose(0, 2, 1, 3)
    Qr4 = jnp.pad(
        Qr.reshape(1, S, H, Dr), ((0, 0), (0, 0), (0, 0), (0, Dh - Dr))
    ).transpose(0, 2, 1, 3)
    Kr3 = Kr.reshape(1, S, Dh)

    O = _attention(Q4, K4, V4, Qr4, Kr3)
    O2 = O.transpose(0, 2, 1, 3).reshape(S, H * Dh)
    out_me = _matmul(O2, Wo)

    out = _out_allgather(out_me)
    return out.reshape(B, S, D)


# device time: 204808 ns/iter; 2.2350x vs baseline; 1.2726x over previous
import jax
import jax.numpy as jnp
from jax import lax
from jax.experimental import pallas as pl
from jax.experimental.pallas import tpu as pltpu

B, S, D = 4, 256, 4096
H, Dh, Dr = 32, 128, 64
DC_HALF = 128
SCALE = (Dh + Dr) ** -0.5
_MESH = pl.DeviceIdType.MESH


def _ring_pos(my_x, my_y):
    return jnp.where(my_x == 0, my_y, 3 - my_y)


def _ring_neighbors(my_x, my_y):
    even = (my_x + my_y) % 2 == 0
    right = (jnp.where(even, my_x, 1 - my_x), jnp.where(even, 1 - my_y, my_y))
    left = (jnp.where(even, 1 - my_x, my_x), jnp.where(even, my_y, 1 - my_y))
    return left, right


def _mm_body(x_ref, w_ref, o_ref, acc_ref):
    @pl.when(pl.program_id(2) == 0)
    def _():
        acc_ref[...] = jnp.zeros_like(acc_ref)

    acc_ref[...] += jnp.dot(
        x_ref[...], w_ref[...], preferred_element_type=jnp.float32
    )

    @pl.when(pl.program_id(2) == pl.num_programs(2) - 1)
    def _():
        o_ref[...] = acc_ref[...]


def _matmul(x, w, bm=1024, bn=1024, bk=1024):
    m, k = x.shape
    _, n = w.shape
    bm, bn, bk = min(bm, m), min(bn, n), min(bk, k)
    return pl.pallas_call(
        _mm_body,
        grid=(m // bm, n // bn, k // bk),
        in_specs=[
            pl.BlockSpec((bm, bk), lambda i, j, kk: (i, kk)),
            pl.BlockSpec((bk, bn), lambda i, j, kk: (kk, j)),
        ],
        out_specs=pl.BlockSpec((bm, bn), lambda i, j, kk: (i, j)),
        out_shape=jax.ShapeDtypeStruct((m, n), jnp.float32),
        scratch_shapes=[pltpu.VMEM((bm, bn), jnp.float32)],
    )(x, w)


_QBN = 1024
_QBK = 1024


def _xchg_copies(cp_ref, wuk_ref, wuv_ref, co_ref, wuko_ref, wuvo_ref,
                 send_sems, recv_sems, q_nbr, off, nbr):
    return [
        pltpu.make_async_remote_copy(
            src_ref=cp_ref.at[q_nbr],
            dst_ref=co_ref.at[:, pl.ds(off, DC_HALF)],
            send_sem=send_sems.at[0], recv_sem=recv_sems.at[0],
            device_id=nbr, device_id_type=_MESH,
        ),
        pltpu.make_async_remote_copy(
            src_ref=wuk_ref,
            dst_ref=wuko_ref.at[pl.ds(off, DC_HALF), :],
            send_sem=send_sems.at[1], recv_sem=recv_sems.at[1],
            device_id=nbr, device_id_type=_MESH,
        ),
        pltpu.make_async_remote_copy(
            src_ref=wuv_ref,
            dst_ref=wuvo_ref.at[pl.ds(off, DC_HALF), :],
            send_sem=send_sems.at[2], recv_sem=recv_sems.at[2],
            device_id=nbr, device_id_type=_MESH,
        ),
    ]


def _xq_body(x_ref, wq_ref, cp_ref, wuk_ref, wuv_ref,
             q_out, co_ref, wuko_ref, wuvo_ref,
             acc_ref, send_sems, recv_sems):
    j = pl.program_id(0)
    k = pl.program_id(1)
    nj = pl.num_programs(0)
    nk = pl.num_programs(1)
    my_x = lax.axis_index("x")
    my_y = lax.axis_index("y")
    nbr = (my_x, 1 - my_y)
    q_me = jnp.where(my_x == 0, my_y, 1 - my_y)
    q_nbr = 1 - q_me
    off = my_y * DC_HALF

    @pl.when((j == 0) & (k == 0))
    def _():
        barrier = pltpu.get_barrier_semaphore()
        pl.semaphore_signal(
            barrier, inc=1, device_id=nbr, device_id_type=_MESH
        )
        pl.semaphore_wait(barrier, 1)
        for cp in _xchg_copies(cp_ref, wuk_ref, wuv_ref, co_ref, wuko_ref,
                               wuvo_ref, send_sems, recv_sems,
                               q_nbr, off, nbr):
            cp.start()
        co_ref[:, pl.ds(off, DC_HALF)] = cp_ref[q_me]
        wuko_ref[pl.ds(off, DC_HALF), :] = wuk_ref[...]
        wuvo_ref[pl.ds(off, DC_HALF), :] = wuv_ref[...]

    @pl.when(k == 0)
    def _():
        acc_ref[...] = jnp.zeros_like(acc_ref)

    acc_ref[...] += jnp.dot(
        x_ref[...], wq_ref[...], preferred_element_type=jnp.float32
    )

    @pl.when(k == nk - 1)
    def _():
        q_out[...] = acc_ref[...]

    @pl.when((j == nj - 1) & (k == nk - 1))
    def _():
        for cp in _xchg_copies(cp_ref, wuk_ref, wuv_ref, co_ref, wuko_ref,
                               wuvo_ref, send_sems, recv_sems,
                               q_nbr, off, nbr):
            cp.wait()


def _q_and_y_exchange(x_me, Wq, c_pair, Wuk, Wuv):
    full = lambda shape: pl.BlockSpec(shape, lambda j, k: (0,) * len(shape))
    return pl.pallas_call(
        _xq_body,
        grid=(D // _QBN, D // _QBK),
        in_specs=[
            pl.BlockSpec((S, _QBK), lambda j, k: (0, k)),
            pl.BlockSpec((_QBK, _QBN), lambda j, k: (k, j)),
            full((2, S, DC_HALF)),
            full((DC_HALF, H * Dh)),
            full((DC_HALF, H * Dh)),
        ],
        out_specs=[
            pl.BlockSpec((S, _QBN), lambda j, k: (0, j)),
            full((S, 2 * DC_HALF)),
            full((2 * DC_HALF, H * Dh)),
            full((2 * DC_HALF, H * Dh)),
        ],
        out_shape=(
            jax.ShapeDtypeStruct((S, D), jnp.float32),
            jax.ShapeDtypeStruct((S, 2 * DC_HALF), jnp.float32),
            jax.ShapeDtypeStruct((2 * DC_HALF, H * Dh), jnp.float32),
            jax.ShapeDtypeStruct((2 * DC_HALF, H * Dh), jnp.float32),
        ),
        scratch_shapes=[
            pltpu.VMEM((S, _QBN), jnp.float32),
            pltpu.SemaphoreType.DMA((3,)),
            pltpu.SemaphoreType.DMA((3,)),
        ],
        compiler_params=pltpu.CompilerParams(collective_id=0),
    )(x_me, Wq, c_pair, Wuk, Wuv)


_OBN = 1024
_OBK = 1024
_HALF_S = S // 2


def _h1_copy(out_ref, jj, p_slot, sems_a, sems_b, base, tgt):
    return pltpu.make_async_remote_copy(
        src_ref=out_ref.at[pl.ds(p_slot * S, S), pl.ds(jj * _OBN, _OBN)],
        dst_ref=out_ref.at[pl.ds(p_slot * S, S), pl.ds(jj * _OBN, _OBN)],
        send_sem=sems_a.at[base + jj], recv_sem=sems_b.at[base + jj],
        device_id=tgt, device_id_type=_MESH,
    )


def _h2_copy(out_ref, jj, slot, row_off, sems_a, sems_b, base, tgt):
    return pltpu.make_async_remote_copy(
        src_ref=out_ref.at[pl.ds(slot * S + row_off, _HALF_S),
                           pl.ds(jj * _OBN, _OBN)],
        dst_ref=out_ref.at[pl.ds(slot * S + row_off, _HALF_S),
                           pl.ds(jj * _OBN, _OBN)],
        send_sem=sems_a.at[base + jj], recv_sem=sems_b.at[base + jj],
        device_id=tgt, device_id_type=_MESH,
    )


def _wo_ag_body(o2_ref, wo_ref, out_ref, acc_ref, send_sems, recv_sems):
    j = pl.program_id(0)
    k = pl.program_id(1)
    nj = pl.num_programs(0)
    nk = pl.num_programs(1)
    my_x = lax.axis_index("x")
    my_y = lax.axis_index("y")
    p = _ring_pos(my_x, my_y)
    left, right = _ring_neighbors(my_x, my_y)
    p_left = (p + 3) % 4
    p_right = (p + 1) % 4

    @pl.when((j == 0) & (k == 0))
    def _():
        barrier = pltpu.get_barrier_semaphore()
        for nb in (left, right):
            pl.semaphore_signal(
                barrier, inc=1, device_id=nb, device_id_type=_MESH
            )
        pl.semaphore_wait(barrier, 2)

    @pl.when(k == 0)
    def _():
        acc_ref[...] = jnp.zeros_like(acc_ref)

    acc_ref[...] += jnp.dot(
        o2_ref[...], wo_ref[...], preferred_element_type=jnp.float32
    )

    @pl.when(k == nk - 1)
    def _():
        out_ref[pl.ds(p * S, S), pl.ds(j * _OBN, _OBN)] = acc_ref[...]
        _h1_copy(out_ref, j, p, send_sems, recv_sems, 0, left).start()
        _h1_copy(out_ref, j, p, send_sems, recv_sems, nj, right).start()

    @pl.when((j == nj - 1) & (k == nk - 1))
    def _():
        for jj in range(nj):
            _h1_copy(out_ref, jj, p_right, send_sems, recv_sems,
                     0, left).wait_recv()
            _h2_copy(out_ref, jj, p_right, 0, send_sems, recv_sems,
                     2 * nj, left).start()
            _h1_copy(out_ref, jj, p_left, send_sems, recv_sems,
                     nj, right).wait_recv()
            _h2_copy(out_ref, jj, p_left, _HALF_S, send_sems, recv_sems,
                     3 * nj, right).start()
        for jj in range(nj):
            _h2_copy(out_ref, jj, p_right, 0, send_sems, recv_sems,
                     2 * nj, left).wait_recv()
            _h2_copy(out_ref, jj, p_left, _HALF_S, send_sems, recv_sems,
                     3 * nj, right).wait_recv()
        for jj in range(nj):
            _h1_copy(out_ref, jj, p, send_sems, recv_sems, 0, left).wait_send()
            _h1_copy(out_ref, jj, p, send_sems, recv_sems, nj, right).wait_send()
            _h2_copy(out_ref, jj, p_right, 0, send_sems, recv_sems,
                     2 * nj, left).wait_send()
            _h2_copy(out_ref, jj, p_left, _HALF_S, send_sems, recv_sems,
                     3 * nj, right).wait_send()


def _wo_and_allgather(O2, Wo):
    nj = (H * Dh) // _OBN
    return pl.pallas_call(
        _wo_ag_body,
        grid=(nj, D // _OBK),
        in_specs=[
            pl.BlockSpec((S, _OBK), lambda j, k: (0, k)),
            pl.BlockSpec((_OBK, _OBN), lambda j, k: (k, j)),
        ],
        out_specs=pl.BlockSpec((B * S, D), lambda j, k: (0, 0)),
        out_shape=jax.ShapeDtypeStruct((B * S, D), jnp.float32),
        scratch_shapes=[
            pltpu.VMEM((S, _OBN), jnp.float32),
            pltpu.SemaphoreType.DMA((4 * nj,)),
            pltpu.SemaphoreType.DMA((4 * nj,)),
        ],
        compiler_params=pltpu.CompilerParams(collective_id=1),
    )(O2, Wo)


def _attn_body(q_ref, k_ref, v_ref, qr_ref, kr_ref, o_ref):
    q = q_ref[...]
    k = k_ref[...]
    v = v_ref[...]
    qr = qr_ref[0, :, :]
    kr = kr_ref[...]
    s = (
        lax.dot_general(q, k, (((1,), (1,)), ((), ())),
                        preferred_element_type=jnp.float32)
        + lax.dot_general(qr, kr, (((1,), (1,)), ((), ())),
                          preferred_element_type=jnp.float32)
    ) * SCALE
    m = jnp.max(s, axis=-1, keepdims=True)
    pr = jnp.exp(s - m)
    pr = pr / jnp.sum(pr, axis=-1, keepdims=True)
    o_ref[...] = jnp.dot(pr, v, preferred_element_type=jnp.float32)


def _attention(Q, K, V, Qr3, Kr):
    cs = pl.BlockSpec((S, Dh), lambda h: (0, h))
    return pl.pallas_call(
        _attn_body,
        grid=(H,),
        in_specs=[
            cs, cs, cs,
            pl.BlockSpec((1, S, Dr), lambda h: (h, 0, 0)),
            pl.BlockSpec((S, Dr), lambda h: (0, 0)),
        ],
        out_specs=cs,
        out_shape=jax.ShapeDtypeStruct((S, H * Dh), jnp.float32),
    )(Q, K, V, Qr3, Kr)


def kernel(x, Wdkv, Wuk, Wuv, Wq, Wqr, Wkr, Wo):
    my_x = lax.axis_index("x")
    my_y = lax.axis_index("y")
    p = _ring_pos(my_x, my_y)

    x_pair = lax.dynamic_slice(x, (2 * my_x, 0, 0), (2, S, D))
    c_pair = _matmul(x_pair.reshape(2 * S, D), Wdkv).reshape(2, S, DC_HALF)

    x_me = lax.dynamic_slice(x, (p, 0, 0), (1, S, D)).reshape(S, D)
    Q, c_me, Wuk_f, Wuv_f = _q_and_y_exchange(x_me, Wq, c_pair, Wuk, Wuv)

    K = _matmul(c_me, Wuk_f)
    V = _matmul(c_me, Wuv_f)
    Qr = _matmul(x_me, Wqr)
    Kr = _matmul(x_me, Wkr)

    Qr3 = Qr.reshape(S, H, Dr).transpose(1, 0, 2)
    O2 = _attention(Q, K, V, Qr3, Kr)

    out = _wo_and_allgather(O2, Wo)
    return out.reshape(B, S, D)


# device time: 194364 ns/iter; 2.3551x vs baseline; 1.0537x over previous
import jax
import jax.numpy as jnp
from jax import lax
from jax.experimental import pallas as pl
from jax.experimental.pallas import tpu as pltpu

B, S, D = 4, 256, 4096
H, Dh, Dr = 32, 128, 64
DC_HALF = 128
N_KV = H * Dh
SCALE = (Dh + Dr) ** -0.5
_MESH = pl.DeviceIdType.MESH


def _ring_pos(my_x, my_y):
    return jnp.where(my_x == 0, my_y, 3 - my_y)


def _ring_neighbors(my_x, my_y):
    even = (my_x + my_y) % 2 == 0
    right = (jnp.where(even, my_x, 1 - my_x), jnp.where(even, 1 - my_y, my_y))
    left = (jnp.where(even, 1 - my_x, my_x), jnp.where(even, my_y, 1 - my_y))
    return left, right


def _mm_body(x_ref, w_ref, o_ref, acc_ref):
    @pl.when(pl.program_id(2) == 0)
    def _():
        acc_ref[...] = jnp.zeros_like(acc_ref)

    acc_ref[...] += jnp.dot(
        x_ref[...], w_ref[...], preferred_element_type=jnp.float32
    )

    @pl.when(pl.program_id(2) == pl.num_programs(2) - 1)
    def _():
        o_ref[...] = acc_ref[...]


def _matmul(x, w, bm=1024, bn=1024, bk=1024):
    m, k = x.shape
    _, n = w.shape
    bm, bn, bk = min(bm, m), min(bn, n), min(bk, k)
    return pl.pallas_call(
        _mm_body,
        grid=(m // bm, n // bn, k // bk),
        in_specs=[
            pl.BlockSpec((bm, bk), lambda i, j, kk: (i, kk)),
            pl.BlockSpec((bk, bn), lambda i, j, kk: (kk, j)),
        ],
        out_specs=pl.BlockSpec((bm, bn), lambda i, j, kk: (i, j)),
        out_shape=jax.ShapeDtypeStruct((m, n), jnp.float32),
        scratch_shapes=[pltpu.VMEM((bm, bn), jnp.float32)],
    )(x, w)


_QBN = 1024
_QBK = 1024
_WSUB = 1024


def _w_sub(jj, my_x):
    col = my_x * (N_KV // 2) + (jj % 2) * _WSUB
    return jj < 2, pl.ds(col, _WSUB)


def _ysend_w(refs, jj, my_y, my_x, sems_s, sems_r, nbr_y):
    wuk_ref, wuv_ref, wuko_ref, wuvo_ref = refs
    is_wuk, cols = _w_sub(jj, my_x)
    src = (wuk_ref if is_wuk else wuv_ref).at[:, cols]
    dst = (wuko_ref if is_wuk else wuvo_ref).at[
        pl.ds(my_y * DC_HALF, DC_HALF), cols]
    return pltpu.make_async_remote_copy(
        src_ref=src, dst_ref=dst,
        send_sem=sems_s.at[jj], recv_sem=sems_r.at[jj],
        device_id=nbr_y, device_id_type=_MESH,
    )


def _xfwd_w(refs, jj, my_y, my_x, sems_s, sems_r, peer_x):
    _, _, wuko_ref, wuvo_ref = refs
    is_wuk, cols = _w_sub(jj, my_x)
    region = (wuko_ref if is_wuk else wuvo_ref).at[
        pl.ds((1 - my_y) * DC_HALF, DC_HALF), cols]
    return pltpu.make_async_remote_copy(
        src_ref=region, dst_ref=region,
        send_sem=sems_s.at[jj], recv_sem=sems_r.at[jj],
        device_id=peer_x, device_id_type=_MESH,
    )


def _c_copy(cp_ref, co_ref, b_nbr, my_y, sems_s, sems_r, nbr_y):
    return pltpu.make_async_remote_copy(
        src_ref=cp_ref.at[pl.ds(b_nbr * S, S), :],
        dst_ref=co_ref.at[:, pl.ds(my_y * DC_HALF, DC_HALF)],
        send_sem=sems_s.at[4], recv_sem=sems_r.at[4],
        device_id=nbr_y, device_id_type=_MESH,
    )


def _xq_body(x_ref, wq_ref, cp_ref, wuk_ref, wuv_ref,
             q_out, co_ref, wuko_ref, wuvo_ref,
             acc_ref, ys_sems, yr_sems, xs_sems, xr_sems):
    j = pl.program_id(0)
    k = pl.program_id(1)
    nj = pl.num_programs(0)
    nk = pl.num_programs(1)
    my_x = lax.axis_index("x")
    my_y = lax.axis_index("y")
    nbr_y = (my_x, 1 - my_y)
    peer_x = (1 - my_x, my_y)
    p = _ring_pos(my_x, my_y)
    b_nbr = 2 * my_x + jnp.where(my_x == 0, 1 - my_y, my_y)
    wrefs = (wuk_ref, wuv_ref, wuko_ref, wuvo_ref)

    @pl.when((j == 0) & (k == 0))
    def _():
        barrier = pltpu.get_barrier_semaphore()
        for nb in (nbr_y, peer_x):
            pl.semaphore_signal(
                barrier, inc=1, device_id=nb, device_id_type=_MESH
            )
        pl.semaphore_wait(barrier, 2)
        _c_copy(cp_ref, co_ref, b_nbr, my_y, ys_sems, yr_sems, nbr_y).start()
        for jj in range(4):
            _ysend_w(wrefs, jj, my_y, my_x, ys_sems, yr_sems, nbr_y).start()
        off = my_y * DC_HALF
        co_ref[:, pl.ds(off, DC_HALF)] = cp_ref[pl.ds(p * S, S), :]
        wuko_ref[pl.ds(off, DC_HALF), :] = wuk_ref[...]
        wuvo_ref[pl.ds(off, DC_HALF), :] = wuv_ref[...]

    @pl.when(k == 0)
    def _():
        acc_ref[...] = jnp.zeros_like(acc_ref)

    acc_ref[...] += jnp.dot(
        x_ref[...], wq_ref[...], preferred_element_type=jnp.float32
    )

    @pl.when(k == nk - 1)
    def _():
        q_out[...] = acc_ref[...]

    for jj in range(4):
        @pl.when((j == jj) & (k == nk - 1))
        def _(jj=jj):
            _ysend_w(wrefs, jj, my_y, my_x, ys_sems, yr_sems,
                     nbr_y).wait_recv()
            _xfwd_w(wrefs, jj, my_y, my_x, xs_sems, xr_sems, peer_x).start()

    @pl.when((j == nj - 1) & (k == nk - 1))
    def _():
        _c_copy(cp_ref, co_ref, b_nbr, my_y, ys_sems, yr_sems,
                nbr_y).wait()
        for jj in range(4):
            _xfwd_w(wrefs, jj, my_y, my_x, xs_sems, xr_sems,
                    peer_x).wait()
            _ysend_w(wrefs, jj, my_y, my_x, ys_sems, yr_sems,
                     nbr_y).wait_send()


def _q_and_y_exchange(x_me, Wq, c_all, Wuk, Wuv):
    full = lambda shape: pl.BlockSpec(shape, lambda j, k: (0,) * len(shape))
    return pl.pallas_call(
        _xq_body,
        grid=(D // _QBN, D // _QBK),
        in_specs=[
            pl.BlockSpec((S, _QBK), lambda j, k: (0, k)),
            pl.BlockSpec((_QBK, _QBN), lambda j, k: (k, j)),
            full((B * S, DC_HALF)),
            full((DC_HALF, N_KV)),
            full((DC_HALF, N_KV)),
        ],
        out_specs=[
            pl.BlockSpec((S, _QBN), lambda j, k: (0, j)),
            full((S, 2 * DC_HALF)),
            full((2 * DC_HALF, N_KV)),
            full((2 * DC_HALF, N_KV)),
        ],
        out_shape=(
            jax.ShapeDtypeStruct((S, D), jnp.float32),
            jax.ShapeDtypeStruct((S, 2 * DC_HALF), jnp.float32),
            jax.ShapeDtypeStruct((2 * DC_HALF, N_KV), jnp.float32),
            jax.ShapeDtypeStruct((2 * DC_HALF, N_KV), jnp.float32),
        ),
        scratch_shapes=[
            pltpu.VMEM((S, _QBN), jnp.float32),
            pltpu.SemaphoreType.DMA((5,)),
            pltpu.SemaphoreType.DMA((5,)),
            pltpu.SemaphoreType.DMA((4,)),
            pltpu.SemaphoreType.DMA((4,)),
        ],
        compiler_params=pltpu.CompilerParams(collective_id=0),
    )(x_me, Wq, c_all, Wuk, Wuv)


_OBN = 1024
_OBK = 1024
_HALF_S = S // 2


def _h1_copy(out_ref, jj, p_slot, sems_a, sems_b, base, tgt):
    return pltpu.make_async_remote_copy(
        src_ref=out_ref.at[pl.ds(p_slot * S, S), pl.ds(jj * _OBN, _OBN)],
        dst_ref=out_ref.at[pl.ds(p_slot * S, S), pl.ds(jj * _OBN, _OBN)],
        send_sem=sems_a.at[base + jj], recv_sem=sems_b.at[base + jj],
        device_id=tgt, device_id_type=_MESH,
    )


def _h2_copy(out_ref, jj, slot, row_off, sems_a, sems_b, base, tgt):
    return pltpu.make_async_remote_copy(
        src_ref=out_ref.at[pl.ds(slot * S + row_off, _HALF_S),
                           pl.ds(jj * _OBN, _OBN)],
        dst_ref=out_ref.at[pl.ds(slot * S + row_off, _HALF_S),
                           pl.ds(jj * _OBN, _OBN)],
        send_sem=sems_a.at[base + jj], recv_sem=sems_b.at[base + jj],
        device_id=tgt, device_id_type=_MESH,
    )


def _wo_ag_body(o2_ref, wo_ref, out_ref, acc_ref, send_sems, recv_sems):
    j = pl.program_id(0)
    k = pl.program_id(1)
    nj = pl.num_programs(0)
    nk = pl.num_programs(1)
    my_x = lax.axis_index("x")
    my_y = lax.axis_index("y")
    p = _ring_pos(my_x, my_y)
    left, right = _ring_neighbors(my_x, my_y)
    p_left = (p + 3) % 4
    p_right = (p + 1) % 4

    @pl.when((j == 0) & (k == 0))
    def _():
        barrier = pltpu.get_barrier_semaphore()
        for nb in (left, right):
            pl.semaphore_signal(
                barrier, inc=1, device_id=nb, device_id_type=_MESH
            )
        pl.semaphore_wait(barrier, 2)

    @pl.when(k == 0)
    def _():
        acc_ref[...] = jnp.zeros_like(acc_ref)

    acc_ref[...] += jnp.dot(
        o2_ref[...], wo_ref[...], preferred_element_type=jnp.float32
    )

    @pl.when(k == nk - 1)
    def _():
        out_ref[pl.ds(p * S, S), pl.ds(j * _OBN, _OBN)] = acc_ref[...]
        _h1_copy(out_ref, j, p, send_sems, recv_sems, 0, left).start()
        _h1_copy(out_ref, j, p, send_sems, recv_sems, nj, right).start()

    @pl.when((k == nk - 1) & (j >= 1))
    def _():
        jj = j - 1
        _h1_copy(out_ref, jj, p_right, send_sems, recv_sems,
                 0, left).wait_recv()
        _h2_copy(out_ref, jj, p_right, 0, send_sems, recv_sems,
                 2 * nj, left).start()
        _h1_copy(out_ref, jj, p_left, send_sems, recv_sems,
                 nj, right).wait_recv()
        _h2_copy(out_ref, jj, p_left, _HALF_S, send_sems, recv_sems,
                 3 * nj, right).start()

    @pl.when((j == nj - 1) & (k == nk - 1))
    def _():
        jj = nj - 1
        _h1_copy(out_ref, jj, p_right, send_sems, recv_sems,
                 0, left).wait_recv()
        _h2_copy(out_ref, jj, p_right, 0, send_sems, recv_sems,
                 2 * nj, left).start()
        _h1_copy(out_ref, jj, p_left, send_sems, recv_sems,
                 nj, right).wait_recv()
        _h2_copy(out_ref, jj, p_left, _HALF_S, send_sems, recv_sems,
                 3 * nj, right).start()
        for jj in range(nj):
            _h2_copy(out_ref, jj, p_right, 0, send_sems, recv_sems,
                     2 * nj, left).wait_recv()
            _h2_copy(out_ref, jj, p_left, _HALF_S, send_sems, recv_sems,
                     3 * nj, right).wait_recv()
        for jj in range(nj):
            _h1_copy(out_ref, jj, p, send_sems, recv_sems, 0, left).wait_send()
            _h1_copy(out_ref, jj, p, send_sems, recv_sems, nj, right).wait_send()
            _h2_copy(out_ref, jj, p_right, 0, send_sems, recv_sems,
                     2 * nj, left).wait_send()
            _h2_copy(out_ref, jj, p_left, _HALF_S, send_sems, recv_sems,
                     3 * nj, right).wait_send()


def _wo_and_allgather(O2, Wo):
    nj = D // _OBN
    return pl.pallas_call(
        _wo_ag_body,
        grid=(nj, (H * Dh) // _OBK),
        in_specs=[
            pl.BlockSpec((S, _OBK), lambda j, k: (0, k)),
            pl.BlockSpec((_OBK, _OBN), lambda j, k: (k, j)),
        ],
        out_specs=pl.BlockSpec((B * S, D), lambda j, k: (0, 0)),
        out_shape=jax.ShapeDtypeStruct((B * S, D), jnp.float32),
        scratch_shapes=[
            pltpu.VMEM((S, _OBN), jnp.float32),
            pltpu.SemaphoreType.DMA((4 * nj,)),
            pltpu.SemaphoreType.DMA((4 * nj,)),
        ],
        compiler_params=pltpu.CompilerParams(collective_id=1),
    )(O2, Wo)


def _attn_body(q_ref, k_ref, v_ref, qr_ref, kr_ref, o_ref):
    q = q_ref[...]
    k = k_ref[...]
    v = v_ref[...]
    qr = qr_ref[0, :, :]
    kr = kr_ref[...]
    s = (
        lax.dot_general(q, k, (((1,), (1,)), ((), ())),
                        preferred_element_type=jnp.float32)
        + lax.dot_general(qr, kr, (((1,), (1,)), ((), ())),
                          preferred_element_type=jnp.float32)
    ) * SCALE
    m = jnp.max(s, axis=-1, keepdims=True)
    pr = jnp.exp(s - m)
    pr = pr / jnp.sum(pr, axis=-1, keepdims=True)
    o_ref[...] = jnp.dot(pr, v, preferred_element_type=jnp.float32)


def _attention(Q, K, V, Qr3, Kr):
    cs = pl.BlockSpec((S, Dh), lambda h: (0, h))
    return pl.pallas_call(
        _attn_body,
        grid=(H,),
        in_specs=[
            cs, cs, cs,
            pl.BlockSpec((1, S, Dr), lambda h: (h, 0, 0)),
            pl.BlockSpec((S, Dr), lambda h: (0, 0)),
        ],
        out_specs=cs,
        out_shape=jax.ShapeDtypeStruct((S, H * Dh), jnp.float32),
    )(Q, K, V, Qr3, Kr)


def kernel(x, Wdkv, Wuk, Wuv, Wq, Wqr, Wkr, Wo):
    my_x = lax.axis_index("x")
    my_y = lax.axis_index("y")
    p = _ring_pos(my_x, my_y)

    x2 = x.reshape(B * S, D)
    c_all = _matmul(x2, Wdkv)

    x_me = lax.dynamic_slice(x, (p, 0, 0), (1, S, D)).reshape(S, D)
    Q, c_me, Wuk_f, Wuv_f = _q_and_y_exchange(x_me, Wq, c_all, Wuk, Wuv)

    K = _matmul(c_me, Wuk_f)
    V = _matmul(c_me, Wuv_f)
    Qr = _matmul(x_me, Wqr)
    Kr = _matmul(x_me, Wkr)

    Qr3 = Qr.reshape(S, H, Dr).transpose(1, 0, 2)
    O2 = _attention(Q, K, V, Qr3, Kr)

    out = _wo_and_allgather(O2, Wo)
    return out.reshape(B, S, D)


# device time: 180979 ns/iter; 2.5293x vs baseline; 1.0740x over previous
import jax
import jax.numpy as jnp
from jax import lax
from jax.experimental import pallas as pl
from jax.experimental.pallas import tpu as pltpu

B, S, D = 4, 256, 4096
H, Dh, Dr = 32, 128, 64
DC_HALF = 128
N_KV = H * Dh
SCALE = (Dh + Dr) ** -0.5
_MESH = pl.DeviceIdType.MESH


def _ring_pos(my_x, my_y):
    return jnp.where(my_x == 0, my_y, 3 - my_y)


def _ring_neighbors(my_x, my_y):
    even = (my_x + my_y) % 2 == 0
    right = (jnp.where(even, my_x, 1 - my_x), jnp.where(even, 1 - my_y, my_y))
    left = (jnp.where(even, 1 - my_x, my_x), jnp.where(even, my_y, 1 - my_y))
    return left, right


def _mm_body(x_ref, w_ref, o_ref, acc_ref):
    @pl.when(pl.program_id(2) == 0)
    def _():
        acc_ref[...] = jnp.zeros_like(acc_ref)

    acc_ref[...] += jnp.dot(
        x_ref[...], w_ref[...], preferred_element_type=jnp.float32
    )

    @pl.when(pl.program_id(2) == pl.num_programs(2) - 1)
    def _():
        o_ref[...] = acc_ref[...]


def _matmul(x, w, bm=1024, bn=1024, bk=1024):
    m, k = x.shape
    _, n = w.shape
    bm, bn, bk = min(bm, m), min(bn, n), min(bk, k)
    return pl.pallas_call(
        _mm_body,
        grid=(m // bm, n // bn, k // bk),
        in_specs=[
            pl.BlockSpec((bm, bk), lambda i, j, kk: (i, kk)),
            pl.BlockSpec((bk, bn), lambda i, j, kk: (kk, j)),
        ],
        out_specs=pl.BlockSpec((bm, bn), lambda i, j, kk: (i, j)),
        out_shape=jax.ShapeDtypeStruct((m, n), jnp.float32),
        scratch_shapes=[pltpu.VMEM((bm, bn), jnp.float32)],
    )(x, w)


def _row_idx():
    return _ring_pos(lax.axis_index("x"), lax.axis_index("y"))


def _matmul_myrow(x_full, w, bn=1024, bk=1024):
    _, k = x_full.shape
    _, n = w.shape
    bn, bk = min(bn, n), min(bk, k)
    return pl.pallas_call(
        _mm_body,
        grid=(1, n // bn, k // bk),
        in_specs=[
            pl.BlockSpec((S, bk), lambda i, j, kk: (_row_idx(), kk)),
            pl.BlockSpec((bk, bn), lambda i, j, kk: (kk, j)),
        ],
        out_specs=pl.BlockSpec((S, bn), lambda i, j, kk: (i, j)),
        out_shape=jax.ShapeDtypeStruct((S, n), jnp.float32),
        scratch_shapes=[pltpu.VMEM((S, bn), jnp.float32)],
    )(x_full, w)


_QBN = 1024
_QBK = 1024
_WSUB = 1024


def _w_sub(jj, my_x):
    col = my_x * (N_KV // 2) + (jj % 2) * _WSUB
    return jj < 2, pl.ds(col, _WSUB)


def _ysend_w(refs, jj, my_y, my_x, sems_s, sems_r, nbr_y):
    wuk_ref, wuv_ref, wuko_ref, wuvo_ref = refs
    is_wuk, cols = _w_sub(jj, my_x)
    src = (wuk_ref if is_wuk else wuv_ref).at[:, cols]
    dst = (wuko_ref if is_wuk else wuvo_ref).at[
        pl.ds(my_y * DC_HALF, DC_HALF), cols]
    return pltpu.make_async_remote_copy(
        src_ref=src, dst_ref=dst,
        send_sem=sems_s.at[jj], recv_sem=sems_r.at[jj],
        device_id=nbr_y, device_id_type=_MESH,
    )


def _xfwd_w(refs, jj, my_y, my_x, sems_s, sems_r, peer_x):
    _, _, wuko_ref, wuvo_ref = refs
    is_wuk, cols = _w_sub(jj, my_x)
    region = (wuko_ref if is_wuk else wuvo_ref).at[
        pl.ds((1 - my_y) * DC_HALF, DC_HALF), cols]
    return pltpu.make_async_remote_copy(
        src_ref=region, dst_ref=region,
        send_sem=sems_s.at[jj], recv_sem=sems_r.at[jj],
        device_id=peer_x, device_id_type=_MESH,
    )


def _c_copy(cp_ref, co_ref, b_nbr, my_y, sems_s, sems_r, nbr_y):
    return pltpu.make_async_remote_copy(
        src_ref=cp_ref.at[pl.ds(b_nbr * S, S), :],
        dst_ref=co_ref.at[:, pl.ds(my_y * DC_HALF, DC_HALF)],
        send_sem=sems_s.at[4], recv_sem=sems_r.at[4],
        device_id=nbr_y, device_id_type=_MESH,
    )


def _xq_body(x_ref, wq_ref, cp_ref, wuk_ref, wuv_ref,
             q_out, co_ref, wuko_ref, wuvo_ref,
             acc_ref, ys_sems, yr_sems, xs_sems, xr_sems):
    j = pl.program_id(0)
    k = pl.program_id(1)
    nj = pl.num_programs(0)
    nk = pl.num_programs(1)
    my_x = lax.axis_index("x")
    my_y = lax.axis_index("y")
    nbr_y = (my_x, 1 - my_y)
    peer_x = (1 - my_x, my_y)
    p = _ring_pos(my_x, my_y)
    b_nbr = 2 * my_x + jnp.where(my_x == 0, 1 - my_y, my_y)
    wrefs = (wuk_ref, wuv_ref, wuko_ref, wuvo_ref)

    @pl.when((j == 0) & (k == 0))
    def _():
        barrier = pltpu.get_barrier_semaphore()
        for nb in (nbr_y, peer_x):
            pl.semaphore_signal(
                barrier, inc=1, device_id=nb, device_id_type=_MESH
            )
        pl.semaphore_wait(barrier, 2)
        _c_copy(cp_ref, co_ref, b_nbr, my_y, ys_sems, yr_sems, nbr_y).start()
        for jj in range(4):
            _ysend_w(wrefs, jj, my_y, my_x, ys_sems, yr_sems, nbr_y).start()
        off = my_y * DC_HALF
        co_ref[:, pl.ds(off, DC_HALF)] = cp_ref[pl.ds(p * S, S), :]
        wuko_ref[pl.ds(off, DC_HALF), :] = wuk_ref[...]
        wuvo_ref[pl.ds(off, DC_HALF), :] = wuv_ref[...]

    @pl.when(k == 0)
    def _():
        acc_ref[...] = jnp.zeros_like(acc_ref)

    acc_ref[...] += jnp.dot(
        x_ref[...], wq_ref[...], preferred_element_type=jnp.float32
    )

    @pl.when(k == nk - 1)
    def _():
        q_out[...] = acc_ref[...]

    for jj in range(4):
        @pl.when((j == jj) & (k == nk - 1))
        def _(jj=jj):
            _ysend_w(wrefs, jj, my_y, my_x, ys_sems, yr_sems,
                     nbr_y).wait_recv()
            _xfwd_w(wrefs, jj, my_y, my_x, xs_sems, xr_sems, peer_x).start()

    @pl.when((j == nj - 1) & (k == nk - 1))
    def _():
        _c_copy(cp_ref, co_ref, b_nbr, my_y, ys_sems, yr_sems,
                nbr_y).wait()
        for jj in range(4):
            _xfwd_w(wrefs, jj, my_y, my_x, xs_sems, xr_sems,
                    peer_x).wait()
            _ysend_w(wrefs, jj, my_y, my_x, ys_sems, yr_sems,
                     nbr_y).wait_send()


def _q_and_y_exchange(x_me, Wq, c_all, Wuk, Wuv):
    full = lambda shape: pl.BlockSpec(shape, lambda j, k: (0,) * len(shape))
    return pl.pallas_call(
        _xq_body,
        grid=(D // _QBN, D // _QBK),
        in_specs=[
            pl.BlockSpec((S, _QBK), lambda j, k: (0, k)),
            pl.BlockSpec((_QBK, _QBN), lambda j, k: (k, j)),
            full((B * S, DC_HALF)),
            full((DC_HALF, N_KV)),
            full((DC_HALF, N_KV)),
        ],
        out_specs=[
            pl.BlockSpec((S, _QBN), lambda j, k: (0, j)),
            full((S, 2 * DC_HALF)),
            full((2 * DC_HALF, N_KV)),
            full((2 * DC_HALF, N_KV)),
        ],
        out_shape=(
            jax.ShapeDtypeStruct((S, D), jnp.float32),
            jax.ShapeDtypeStruct((S, 2 * DC_HALF), jnp.float32),
            jax.ShapeDtypeStruct((2 * DC_HALF, N_KV), jnp.float32),
            jax.ShapeDtypeStruct((2 * DC_HALF, N_KV), jnp.float32),
        ),
        scratch_shapes=[
            pltpu.VMEM((S, _QBN), jnp.float32),
            pltpu.SemaphoreType.DMA((5,)),
            pltpu.SemaphoreType.DMA((5,)),
            pltpu.SemaphoreType.DMA((4,)),
            pltpu.SemaphoreType.DMA((4,)),
        ],
        compiler_params=pltpu.CompilerParams(collective_id=0),
    )(x_me, Wq, c_all, Wuk, Wuv)


_OBN = 1024
_OBK = 1024
_HALF_S = S // 2


def _h1_copy(out_ref, jj, p_slot, sems_a, sems_b, base, tgt):
    return pltpu.make_async_remote_copy(
        src_ref=out_ref.at[pl.ds(p_slot * S, S), pl.ds(jj * _OBN, _OBN)],
        dst_ref=out_ref.at[pl.ds(p_slot * S, S), pl.ds(jj * _OBN, _OBN)],
        send_sem=sems_a.at[base + jj], recv_sem=sems_b.at[base + jj],
        device_id=tgt, device_id_type=_MESH,
    )


def _h2_copy(out_ref, jj, slot, row_off, sems_a, sems_b, base, tgt):
    return pltpu.make_async_remote_copy(
        src_ref=out_ref.at[pl.ds(slot * S + row_off, _HALF_S),
                           pl.ds(jj * _OBN, _OBN)],
        dst_ref=out_ref.at[pl.ds(slot * S + row_off, _HALF_S),
                           pl.ds(jj * _OBN, _OBN)],
        send_sem=sems_a.at[base + jj], recv_sem=sems_b.at[base + jj],
        device_id=tgt, device_id_type=_MESH,
    )


def _wo_ag_body(o2_ref, wo_ref, out_ref, acc_ref, send_sems, recv_sems):
    j = pl.program_id(0)
    k = pl.program_id(1)
    nj = pl.num_programs(0)
    nk = pl.num_programs(1)
    my_x = lax.axis_index("x")
    my_y = lax.axis_index("y")
    p = _ring_pos(my_x, my_y)
    left, right = _ring_neighbors(my_x, my_y)
    p_left = (p + 3) % 4
    p_right = (p + 1) % 4

    @pl.when((j == 0) & (k == 0))
    def _():
        barrier = pltpu.get_barrier_semaphore()
        for nb in (left, right):
            pl.semaphore_signal(
                barrier, inc=1, device_id=nb, device_id_type=_MESH
            )
        pl.semaphore_wait(barrier, 2)

    @pl.when(k == 0)
    def _():
        acc_ref[...] = jnp.zeros_like(acc_ref)

    acc_ref[...] += jnp.dot(
        o2_ref[...], wo_ref[...], preferred_element_type=jnp.float32
    )

    @pl.when(k == nk - 1)
    def _():
        out_ref[pl.ds(p * S, S), pl.ds(j * _OBN, _OBN)] = acc_ref[...]
        _h1_copy(out_ref, j, p, send_sems, recv_sems, 0, left).start()
        _h1_copy(out_ref, j, p, send_sems, recv_sems, nj, right).start()

    @pl.when((k == nk - 1) & (j >= 1))
    def _():
        jj = j - 1
        _h1_copy(out_ref, jj, p_right, send_sems, recv_sems,
                 0, left).wait_recv()
        _h2_copy(out_ref, jj, p_right, 0, send_sems, recv_sems,
                 2 * nj, left).start()
        _h1_copy(out_ref, jj, p_left, send_sems, recv_sems,
                 nj, right).wait_recv()
        _h2_copy(out_ref, jj, p_left, _HALF_S, send_sems, recv_sems,
                 3 * nj, right).start()

    @pl.when((j == nj - 1) & (k == nk - 1))
    def _():
        jj = nj - 1
        _h1_copy(out_ref, jj, p_right, send_sems, recv_sems,
                 0, left).wait_recv()
        _h2_copy(out_ref, jj, p_right, 0, send_sems, recv_sems,
                 2 * nj, left).start()
        _h1_copy(out_ref, jj, p_left, send_sems, recv_sems,
                 nj, right).wait_recv()
        _h2_copy(out_ref, jj, p_left, _HALF_S, send_sems, recv_sems,
                 3 * nj, right).start()
        for jj in range(nj):
            _h2_copy(out_ref, jj, p_right, 0, send_sems, recv_sems,
                     2 * nj, left).wait_recv()
            _h2_copy(out_ref, jj, p_left, _HALF_S, send_sems, recv_sems,
                     3 * nj, right).wait_recv()
        for jj in range(nj):
            _h1_copy(out_ref, jj, p, send_sems, recv_sems, 0, left).wait_send()
            _h1_copy(out_ref, jj, p, send_sems, recv_sems, nj, right).wait_send()
            _h2_copy(out_ref, jj, p_right, 0, send_sems, recv_sems,
                     2 * nj, left).wait_send()
            _h2_copy(out_ref, jj, p_left, _HALF_S, send_sems, recv_sems,
                     3 * nj, right).wait_send()


def _wo_and_allgather(O2, Wo):
    nj = D // _OBN
    return pl.pallas_call(
        _wo_ag_body,
        grid=(nj, (H * Dh) // _OBK),
        in_specs=[
            pl.BlockSpec((S, _OBK), lambda j, k: (0, k)),
            pl.BlockSpec((_OBK, _OBN), lambda j, k: (k, j)),
        ],
        out_specs=pl.BlockSpec((B * S, D), lambda j, k: (0, 0)),
        out_shape=jax.ShapeDtypeStruct((B * S, D), jnp.float32),
        scratch_shapes=[
            pltpu.VMEM((S, _OBN), jnp.float32),
            pltpu.SemaphoreType.DMA((4 * nj,)),
            pltpu.SemaphoreType.DMA((4 * nj,)),
        ],
        compiler_params=pltpu.CompilerParams(collective_id=1),
    )(O2, Wo)


def _attn_body(q_ref, k_ref, v_ref, qr_ref, kr_ref, o_ref):
    h = pl.program_id(0)
    q = q_ref[...]
    k = k_ref[...]
    v = v_ref[...]
    qr_blk = qr_ref[...]
    qr = jnp.where(h % 2 == 0, qr_blk[:, :Dr], qr_blk[:, Dr:])
    kr = kr_ref[...]
    s = (
        lax.dot_general(q, k, (((1,), (1,)), ((), ())),
                        preferred_element_type=jnp.float32)
        + lax.dot_general(qr, kr, (((1,), (1,)), ((), ())),
                          preferred_element_type=jnp.float32)
    )
    pr = jnp.exp(s * SCALE)
    pr = pr * (1.0 / jnp.sum(pr, axis=-1, keepdims=True))
    o_ref[...] = jnp.dot(pr, v, preferred_element_type=jnp.float32)


def _attention(Q, K, V, Qr, Kr):
    cs = pl.BlockSpec((S, Dh), lambda h: (0, h))
    return pl.pallas_call(
        _attn_body,
        grid=(H,),
        in_specs=[
            cs, cs, cs,
            pl.BlockSpec((S, 2 * Dr), lambda h: (0, h // 2)),
            pl.BlockSpec((S, Dr), lambda h: (0, 0)),
        ],
        out_specs=cs,
        out_shape=jax.ShapeDtypeStruct((S, H * Dh), jnp.float32),
    )(Q, K, V, Qr, Kr)


def kernel(x, Wdkv, Wuk, Wuv, Wq, Wqr, Wkr, Wo):
    my_x = lax.axis_index("x")
    my_y = lax.axis_index("y")
    p = _ring_pos(my_x, my_y)

    x2 = x.reshape(B * S, D)
    c_all = _matmul(x2, Wdkv)

    x_me = lax.dynamic_slice(x, (p, 0, 0), (1, S, D)).reshape(S, D)
    Q, c_me, Wuk_f, Wuv_f = _q_and_y_exchange(x_me, Wq, c_all, Wuk, Wuv)

    K = _matmul(c_me, Wuk_f)
    V = _matmul(c_me, Wuv_f)
    Qr = _matmul(x_me, Wqr)
    Kr = _matmul(x_me, Wkr)

    O2 = _attention(Q, K, V, Qr, Kr)

    out = _wo_and_allgather(O2, Wo)
    return out.reshape(B, S, D)
